# Initial kernel scaffold; baseline (speedup 1.0000x reference)
import jax
import jax.numpy as jnp
from jax.experimental import pallas as pl


def kernel(x, target):
    raise NotImplementedError("write your pallas kernel here")


# speedup vs baseline: 2.5350x; 2.5350x over previous
import functools
import math

import jax
import jax.numpy as jnp
from jax import lax
from jax.experimental import pallas as pl
from jax.experimental.pallas import tpu as pltpu
from jax.experimental.pallas import tpu_sc as plsc

SIZE = 32000
N_TOK = 4096
SMOOTHING = 0.1
CONFIDENCE = 1.0 - SMOOTHING
U = SMOOTHING / (SIZE - 2)
K_CONST = (SIZE - 2) * U * math.log(U) + CONFIDENCE * math.log(CONFIDENCE)

LANES = 16
NW = 32
B_PER_W = N_TOK // NW
CHUNKS = B_PER_W // LANES
ROW_W = LANES
N_GROWS = (N_TOK * SIZE) // ROW_W
COLS_PER_ROW = SIZE // ROW_W

BN = 640
GRID_J = SIZE // BN


def _tc_body(t_ref, x_ref, out_ref):
    j = pl.program_id(0)

    @pl.when(j == 0)
    def _init():
        out_ref[...] = jnp.zeros_like(out_ref)

    tile = x_ref[...]
    mask = (t_ref[...] != 0).astype(jnp.float32)
    rowsum = jnp.sum(tile, axis=1, keepdims=True)
    val = jnp.float32(-U) * jnp.sum(rowsum * mask)
    extra = (jnp.float32(U) * jnp.sum(tile[:, 0:1] * mask)
             + jnp.float32(K_CONST) * jnp.sum(mask))
    val = val + jnp.where(j == 0, extra, jnp.float32(0.0))
    out_ref[...] += val


def _tc_reduce(x, t2d):
    return pl.pallas_call(
        _tc_body,
        grid=(GRID_J,),
        in_specs=[
            pl.BlockSpec((N_TOK, 1), lambda j: (0, 0)),
            pl.BlockSpec((N_TOK, BN), lambda j: (0, j)),
        ],
        out_specs=pl.BlockSpec((1, 1), lambda j: (0, 0)),
        out_shape=jax.ShapeDtypeStruct((1, 1), jnp.float32),
    )(t2d, x)


def _sc_gather_body(xf_hbm, tgt_hbm, out_hbm, tgt_v, idx_v, vals_v, acc_v, sem):
    wid = lax.axis_index("s") * 2 + lax.axis_index("c")
    base = wid * B_PER_W
    pltpu.sync_copy(tgt_hbm.at[pl.ds(base, B_PER_W)], tgt_v)
    for k in range(CHUNKS):
        tv = tgt_v[pl.ds(k * LANES, LANES)]
        gi = (base + k * LANES) + lax.iota(jnp.int32, LANES)
        idx_v[pl.ds(k * LANES, LANES)] = gi * SIZE + tv
    pltpu.async_copy(xf_hbm.at[idx_v], vals_v, sem).wait()
    acc = jnp.zeros((LANES,), jnp.float32)
    for k in range(CHUNKS):
        tv = tgt_v[pl.ds(k * LANES, LANES)]
        val = vals_v[pl.ds(k * LANES, LANES)]
        acc = acc + jnp.where(tv != 0, val, jnp.float32(0.0))
    acc_v[...] = acc
    pltpu.sync_copy(acc_v, out_hbm.at[wid])


@functools.lru_cache(maxsize=1)
def _make_sc_gather():
    return pl.kernel(
        _sc_gather_body,
        out_type=jax.ShapeDtypeStruct((NW, LANES), jnp.float32),
        scratch_types=[
            pltpu.VMEM((B_PER_W,), jnp.int32),
            pltpu.VMEM((B_PER_W,), jnp.int32),
            pltpu.VMEM((B_PER_W,), jnp.float32),
            pltpu.VMEM((LANES,), jnp.float32),
            pltpu.SemaphoreType.DMA,
        ],
        mesh=plsc.VectorSubcoreMesh(core_axis_name="c", subcore_axis_name="s"),
    )


@jax.jit
def kernel(x, target):
    xf = x.reshape(N_TOK * SIZE)
    sc_part = _make_sc_gather()(xf, target)
    tc_part = _tc_reduce(x, target.reshape(N_TOK, 1))
    gather_sum = jnp.sum(sc_part)
    return tc_part[0, 0] - jnp.float32(CONFIDENCE - U) * gather_sum
